# baseline (speedup 1.0000x reference)
import jax
import jax.numpy as jnp
import numpy as np
from jax import lax
from jax.experimental import pallas as pl
from jax.experimental.pallas import tpu as pltpu, tpu_sc as plsc

_TOPK = 100
_THRESH = np.float32(0.5)
_NEG = np.float32(-1e30)
_L = 16
_NS = 16
_BIG = np.int32(1 << 30)


def _nms_body(x1h, y1h, x2h, y2h, sh, outh,
              x1v, y1v, x2v, y2v, sv, oscv, pubv, candv, outv, shv):
    cid = lax.axis_index("c")
    sid = lax.axis_index("s")
    shard = x1v.shape[0]
    nch = shard // _L
    base = sid * shard
    iota = lax.broadcasted_iota(jnp.int32, (_L,), 0)
    zeros_i = jnp.zeros((_L,), jnp.int32)

    pltpu.sync_copy(x1h.at[pl.ds(base, shard)], x1v)
    pltpu.sync_copy(y1h.at[pl.ds(base, shard)], y1v)
    pltpu.sync_copy(x2h.at[pl.ds(base, shard)], x2v)
    pltpu.sync_copy(y2h.at[pl.ds(base, shard)], y2v)
    pltpu.sync_copy(sh.at[pl.ds(base, shard)], sv)
    pltpu.sync_copy(sh.at[pl.ds(base, shard)], oscv)

    def vmax(v):
        for shf in (8, 4, 2, 1):
            p = jnp.take_along_axis(v, iota ^ np.int32(shf), axis=0,
                                    mode="promise_in_bounds")
            v = jnp.maximum(v, p)
        return v

    def vmin_i(v):
        for shf in (8, 4, 2, 1):
            p = jnp.take_along_axis(v, iota ^ np.int32(shf), axis=0,
                                    mode="promise_in_bounds")
            v = jnp.minimum(v, p)
        return v

    def sweep(bx1, by1, bx2, by2, barea):
        def sup_chunk(j, vm):
            sl = pl.ds(j * _L, _L)
            cx1 = x1v[sl]
            cy1 = y1v[sl]
            cx2 = x2v[sl]
            cy2 = y2v[sl]
            cs = sv[sl]
            ca = (cx2 - cx1) * (cy2 - cy1)
            xx1 = jnp.maximum(bx1, cx1)
            yy1 = jnp.maximum(by1, cy1)
            xx2 = jnp.minimum(bx2, cx2)
            yy2 = jnp.minimum(by2, cy2)
            w = jnp.maximum(xx2 - xx1, np.float32(0.0))
            h = jnp.maximum(yy2 - yy1, np.float32(0.0))
            inter = w * h
            denom = barea + ca - inter + np.float32(1e-9)
            sup = (inter + inter) > denom
            ns = jnp.where(sup, _NEG, cs)
            sv[sl] = ns
            return jnp.maximum(vm, ns)

        vm = lax.fori_loop(0, nch, sup_chunk,
                           jnp.full((_L,), -np.float32(np.inf), jnp.float32),
                           unroll=2)
        mv = vmax(vm)

        def idx_chunk(j, iv):
            cs = sv[pl.ds(j * _L, _L)]
            return jnp.minimum(iv, jnp.where(cs == mv, iota + j * _L, _BIG))
        iv = lax.fori_loop(0, nch, idx_chunk, jnp.full((_L,), _BIG, jnp.int32),
                           unroll=4)
        li = vmin_i(iv)
        return mv, li

    def publish(mv, li, buf):
        gx1 = plsc.load_gather(x1v, [li])
        gy1 = plsc.load_gather(y1v, [li])
        gx2 = plsc.load_gather(x2v, [li])
        gy2 = plsc.load_gather(y2v, [li])
        gos = plsc.load_gather(oscv, [li])
        pub = mv
        pub = jnp.where(iota == 1, gx1, pub)
        pub = jnp.where(iota == 2, gy1, pub)
        pub = jnp.where(iota == 3, gx2, pub)
        pub = jnp.where(iota == 4, gy2, pub)
        pub = jnp.where(iota == 5, gos, pub)
        pubv[...] = pub
        pltpu.sync_copy(pubv, shv.at[buf, sid])

    z = jnp.zeros((_L,), jnp.float32)
    mv, li = sweep(z, z, z, z, z)
    publish(mv, li, 0)
    plsc.subcore_barrier()

    def round_body(t, carry):
        buf = lax.rem(t, 2)
        pltpu.sync_copy(shv.at[buf], candv)

        vals = plsc.load_gather(candv, [iota, zeros_i])
        mg = vmax(vals)
        wstar = vmin_i(jnp.where(vals == mg, iota, np.int32(_NS)))

        def fld(f):
            return plsc.load_gather(candv, [wstar, jnp.full((_L,), f, jnp.int32)])
        bx1 = fld(1)
        by1 = fld(2)
        bx2 = fld(3)
        by2 = fld(4)
        bos = fld(5)
        barea = (bx2 - bx1) * (by2 - by1)

        @pl.when(jnp.logical_and(cid == 0, sid == 0))
        def _():
            row = jnp.where(iota == 0, bx1, jnp.zeros((_L,), jnp.float32))
            row = jnp.where(iota == 1, by1, row)
            row = jnp.where(iota == 2, bx2, row)
            row = jnp.where(iota == 3, by2, row)
            row = jnp.where(iota == 4, bos, row)
            plsc.store_scatter(outv, [jnp.full((_L,), t, jnp.int32), iota], row)

        mv, li = sweep(bx1, by1, bx2, by2, barea)
        publish(mv, li, 1 - buf)
        plsc.subcore_barrier()
        return carry

    lax.fori_loop(0, _TOPK, round_body, np.int32(0))

    @pl.when(jnp.logical_and(cid == 0, sid == 0))
    def _():
        pltpu.sync_copy(outv, outh)


@jax.jit
def kernel(x):
    n = x.shape[0]
    npad = ((n + _L * _NS - 1) // (_L * _NS)) * (_L * _NS)
    shard = npad // _NS
    pad = npad - n
    x = x.astype(jnp.float32)
    x1 = jnp.concatenate([x[:, 0], jnp.zeros((pad,), jnp.float32)])
    y1 = jnp.concatenate([x[:, 1], jnp.zeros((pad,), jnp.float32)])
    x2 = jnp.concatenate([x[:, 2], jnp.zeros((pad,), jnp.float32)])
    y2 = jnp.concatenate([x[:, 3], jnp.zeros((pad,), jnp.float32)])
    s = jnp.concatenate([x[:, 4], jnp.full((pad,), _NEG, jnp.float32)])

    mesh = plsc.VectorSubcoreMesh(core_axis_name="c", subcore_axis_name="s",
                                  num_cores=2, num_subcores=_NS)
    f = pl.kernel(
        _nms_body,
        out_type=jax.ShapeDtypeStruct((_TOPK, _L), jnp.float32),
        mesh=mesh,
        compiler_params=pltpu.CompilerParams(needs_layout_passes=False,
                                             use_tc_tiling_on_sc=False),
        scratch_types=[
            pltpu.VMEM((shard,), jnp.float32),
            pltpu.VMEM((shard,), jnp.float32),
            pltpu.VMEM((shard,), jnp.float32),
            pltpu.VMEM((shard,), jnp.float32),
            pltpu.VMEM((shard,), jnp.float32),
            pltpu.VMEM((shard,), jnp.float32),
            pltpu.VMEM((_L,), jnp.float32),
            pltpu.VMEM((_NS, _L), jnp.float32),
            pltpu.VMEM((_TOPK, _L), jnp.float32),
            pltpu.VMEM_SHARED((2, _NS, _L), jnp.float32),
        ],
    )
    out16 = f(x1, y1, x2, y2, s)
    return out16[:, :5]


# speedup vs baseline: 1.6775x; 1.6775x over previous
import jax
import jax.numpy as jnp
import numpy as np
from jax import lax
from jax.experimental import pallas as pl
from jax.experimental.pallas import tpu as pltpu, tpu_sc as plsc

_TOPK = 100
_THRESH = np.float32(0.5)
_NEG = np.float32(-1e30)
_L = 16
_NS = 16
_BIG = np.int32(1 << 30)


def _nms_body(x1h, y1h, x2h, y2h, sh, outh,
              x1v, y1v, x2v, y2v, sv, oscv, pubv, candv, outv, shv):
    cid = lax.axis_index("c")
    sid = lax.axis_index("s")
    shard = x1v.shape[0]
    nch = shard // _L
    base = sid * shard
    iota = lax.broadcasted_iota(jnp.int32, (_L,), 0)
    zeros_i = jnp.zeros((_L,), jnp.int32)

    pltpu.sync_copy(x1h.at[pl.ds(base, shard)], x1v)
    pltpu.sync_copy(y1h.at[pl.ds(base, shard)], y1v)
    pltpu.sync_copy(x2h.at[pl.ds(base, shard)], x2v)
    pltpu.sync_copy(y2h.at[pl.ds(base, shard)], y2v)
    pltpu.sync_copy(sh.at[pl.ds(base, shard)], sv)
    pltpu.sync_copy(sh.at[pl.ds(base, shard)], oscv)

    def vmax(v):
        for shf in (8, 4, 2, 1):
            p = jnp.take_along_axis(v, iota ^ np.int32(shf), axis=0,
                                    mode="promise_in_bounds")
            v = jnp.maximum(v, p)
        return v

    def vmin_i(v):
        for shf in (8, 4, 2, 1):
            p = jnp.take_along_axis(v, iota ^ np.int32(shf), axis=0,
                                    mode="promise_in_bounds")
            v = jnp.minimum(v, p)
        return v

    def sweep(bx1, by1, bx2, by2, barea):
        def sup_chunk(j, vm):
            sl = pl.ds(j * _L, _L)
            cx1 = x1v[sl]
            cy1 = y1v[sl]
            cx2 = x2v[sl]
            cy2 = y2v[sl]
            cs = sv[sl]
            ca = (cx2 - cx1) * (cy2 - cy1)
            xx1 = jnp.maximum(bx1, cx1)
            yy1 = jnp.maximum(by1, cy1)
            xx2 = jnp.minimum(bx2, cx2)
            yy2 = jnp.minimum(by2, cy2)
            w = jnp.maximum(xx2 - xx1, np.float32(0.0))
            h = jnp.maximum(yy2 - yy1, np.float32(0.0))
            inter = w * h
            denom = barea + ca - inter + np.float32(1e-9)
            sup = (inter + inter) > denom
            ns = jnp.where(sup, _NEG, cs)
            sv[sl] = ns
            return jnp.maximum(vm, ns)

        vm = lax.fori_loop(0, nch, sup_chunk,
                           jnp.full((_L,), -np.float32(np.inf), jnp.float32))
        mv = vmax(vm)

        def idx_chunk(j, iv):
            cs = sv[pl.ds(j * _L, _L)]
            return jnp.minimum(iv, jnp.where(cs == mv, iota + j * _L, _BIG))
        iv = lax.fori_loop(0, nch, idx_chunk, jnp.full((_L,), _BIG, jnp.int32),
                           unroll=8)
        li = vmin_i(iv)
        return mv, li

    def publish(mv, li, buf):
        gx1 = plsc.load_gather(x1v, [li])
        gy1 = plsc.load_gather(y1v, [li])
        gx2 = plsc.load_gather(x2v, [li])
        gy2 = plsc.load_gather(y2v, [li])
        gos = plsc.load_gather(oscv, [li])
        pub = mv
        pub = jnp.where(iota == 1, gx1, pub)
        pub = jnp.where(iota == 2, gy1, pub)
        pub = jnp.where(iota == 3, gx2, pub)
        pub = jnp.where(iota == 4, gy2, pub)
        pub = jnp.where(iota == 5, gos, pub)
        pubv[...] = pub
        pltpu.sync_copy(pubv, shv.at[buf, sid])

    z = jnp.zeros((_L,), jnp.float32)
    mv, li = sweep(z, z, z, z, z)
    publish(mv, li, 0)
    plsc.subcore_barrier()

    def round_body(t, carry):
        buf = lax.rem(t, 2)
        pltpu.sync_copy(shv.at[buf], candv)

        vals = plsc.load_gather(candv, [iota, zeros_i])
        mg = vmax(vals)
        wstar = plsc.all_reduce_ffs(vals == mg)

        def fld(f):
            return plsc.load_gather(candv, [wstar, jnp.full((_L,), f, jnp.int32)])
        bx1 = fld(1)
        by1 = fld(2)
        bx2 = fld(3)
        by2 = fld(4)
        bos = fld(5)
        barea = (bx2 - bx1) * (by2 - by1)

        @pl.when(jnp.logical_and(cid == 0, sid == 0))
        def _():
            row = jnp.where(iota == 0, bx1, jnp.zeros((_L,), jnp.float32))
            row = jnp.where(iota == 1, by1, row)
            row = jnp.where(iota == 2, bx2, row)
            row = jnp.where(iota == 3, by2, row)
            row = jnp.where(iota == 4, bos, row)
            plsc.store_scatter(outv, [jnp.full((_L,), t, jnp.int32), iota], row)

        mv, li = sweep(bx1, by1, bx2, by2, barea)
        publish(mv, li, 1 - buf)
        plsc.subcore_barrier()
        return carry

    lax.fori_loop(0, _TOPK, round_body, np.int32(0))

    @pl.when(jnp.logical_and(cid == 0, sid == 0))
    def _():
        pltpu.sync_copy(outv, outh)


@jax.jit
def kernel(x):
    n = x.shape[0]
    npad = ((n + _L * _NS - 1) // (_L * _NS)) * (_L * _NS)
    shard = npad // _NS
    pad = npad - n
    x = x.astype(jnp.float32)
    x1 = jnp.concatenate([x[:, 0], jnp.zeros((pad,), jnp.float32)])
    y1 = jnp.concatenate([x[:, 1], jnp.zeros((pad,), jnp.float32)])
    x2 = jnp.concatenate([x[:, 2], jnp.zeros((pad,), jnp.float32)])
    y2 = jnp.concatenate([x[:, 3], jnp.zeros((pad,), jnp.float32)])
    s = jnp.concatenate([x[:, 4], jnp.full((pad,), _NEG, jnp.float32)])

    mesh = plsc.VectorSubcoreMesh(core_axis_name="c", subcore_axis_name="s",
                                  num_cores=2, num_subcores=_NS)
    f = pl.kernel(
        _nms_body,
        out_type=jax.ShapeDtypeStruct((_TOPK, _L), jnp.float32),
        mesh=mesh,
        compiler_params=pltpu.CompilerParams(needs_layout_passes=False,
                                             use_tc_tiling_on_sc=False),
        scratch_types=[
            pltpu.VMEM((shard,), jnp.float32),
            pltpu.VMEM((shard,), jnp.float32),
            pltpu.VMEM((shard,), jnp.float32),
            pltpu.VMEM((shard,), jnp.float32),
            pltpu.VMEM((shard,), jnp.float32),
            pltpu.VMEM((shard,), jnp.float32),
            pltpu.VMEM((_L,), jnp.float32),
            pltpu.VMEM((_NS, _L), jnp.float32),
            pltpu.VMEM((_TOPK, _L), jnp.float32),
            pltpu.VMEM_SHARED((2, _NS, _L), jnp.float32),
        ],
    )
    out16 = f(x1, y1, x2, y2, s)
    return out16[:, :5]


# speedup vs baseline: 1.6844x; 1.0042x over previous
import jax
import jax.numpy as jnp
import numpy as np
from jax import lax
from jax.experimental import pallas as pl
from jax.experimental.pallas import tpu as pltpu, tpu_sc as plsc

_TOPK = 100
_THRESH = np.float32(0.5)
_NEG = np.float32(-1e30)
_L = 16
_NS = 16
_BIG = np.int32(1 << 30)


def _nms_body(x1h, y1h, x2h, y2h, sh, outh,
              x1v, y1v, x2v, y2v, sv, oscv, pubv, candv, outv, shv):
    cid = lax.axis_index("c")
    sid = lax.axis_index("s")
    shard = x1v.shape[0]
    nch = shard // _L
    base = sid * shard
    iota = lax.broadcasted_iota(jnp.int32, (_L,), 0)
    zeros_i = jnp.zeros((_L,), jnp.int32)

    pltpu.sync_copy(x1h.at[pl.ds(base, shard)], x1v)
    pltpu.sync_copy(y1h.at[pl.ds(base, shard)], y1v)
    pltpu.sync_copy(x2h.at[pl.ds(base, shard)], x2v)
    pltpu.sync_copy(y2h.at[pl.ds(base, shard)], y2v)
    pltpu.sync_copy(sh.at[pl.ds(base, shard)], sv)
    pltpu.sync_copy(sh.at[pl.ds(base, shard)], oscv)

    def vmax(v):
        for shf in (8, 4, 2, 1):
            p = jnp.take_along_axis(v, iota ^ np.int32(shf), axis=0,
                                    mode="promise_in_bounds")
            v = jnp.maximum(v, p)
        return v

    def vmin_i(v):
        for shf in (8, 4, 2, 1):
            p = jnp.take_along_axis(v, iota ^ np.int32(shf), axis=0,
                                    mode="promise_in_bounds")
            v = jnp.minimum(v, p)
        return v

    def sweep(bx1, by1, bx2, by2, barea):
        def sup_chunk(j, vm):
            sl = pl.ds(j * _L, _L)
            cx1 = x1v[sl]
            cy1 = y1v[sl]
            cx2 = x2v[sl]
            cy2 = y2v[sl]
            cs = sv[sl]
            ca = (cx2 - cx1) * (cy2 - cy1)
            xx1 = jnp.maximum(bx1, cx1)
            yy1 = jnp.maximum(by1, cy1)
            xx2 = jnp.minimum(bx2, cx2)
            yy2 = jnp.minimum(by2, cy2)
            w = jnp.maximum(xx2 - xx1, np.float32(0.0))
            h = jnp.maximum(yy2 - yy1, np.float32(0.0))
            inter = w * h
            denom = barea + ca - inter + np.float32(1e-9)
            sup = (inter + inter) > denom
            ns = jnp.where(sup, _NEG, cs)
            sv[sl] = ns
            return jnp.maximum(vm, ns)

        vm = lax.fori_loop(0, nch, sup_chunk,
                           jnp.full((_L,), -np.float32(np.inf), jnp.float32))
        mv = vmax(vm)

        def idx_chunk(j, iv):
            cs = sv[pl.ds(j * _L, _L)]
            return jnp.minimum(iv, jnp.where(cs == mv, iota + j * _L, _BIG))
        iv = lax.fori_loop(0, nch, idx_chunk, jnp.full((_L,), _BIG, jnp.int32),
                           unroll=8)
        li = vmin_i(iv)
        return mv, li

    def publish(mv, li, buf):
        gx1 = plsc.load_gather(x1v, [li])
        gy1 = plsc.load_gather(y1v, [li])
        gx2 = plsc.load_gather(x2v, [li])
        gy2 = plsc.load_gather(y2v, [li])
        gos = plsc.load_gather(oscv, [li])
        pub = mv
        pub = jnp.where(iota == 1, gx1, pub)
        pub = jnp.where(iota == 2, gy1, pub)
        pub = jnp.where(iota == 3, gx2, pub)
        pub = jnp.where(iota == 4, gy2, pub)
        pub = jnp.where(iota == 5, gos, pub)
        pubv[...] = pub
        pltpu.sync_copy(pubv, shv.at[buf, sid])

    z = jnp.zeros((_L,), jnp.float32)
    mv, li = sweep(z, z, z, z, z)
    publish(mv, li, 0)
    plsc.subcore_barrier()

    def round_body(t, carry):
        buf = lax.rem(t, 2)
        pltpu.sync_copy(shv.at[buf], candv)

        vals = plsc.load_gather(candv, [iota, zeros_i])
        mg = vmax(vals)
        wstar = plsc.all_reduce_ffs(vals == mg)

        def fld(f):
            return plsc.load_gather(candv, [wstar, jnp.full((_L,), f, jnp.int32)])
        bx1 = fld(1)
        by1 = fld(2)
        bx2 = fld(3)
        by2 = fld(4)
        bos = fld(5)
        barea = (bx2 - bx1) * (by2 - by1)

        @pl.when(jnp.logical_and(cid == 0, sid == 0))
        def _():
            row = jnp.where(iota == 0, bx1, jnp.zeros((_L,), jnp.float32))
            row = jnp.where(iota == 1, by1, row)
            row = jnp.where(iota == 2, bx2, row)
            row = jnp.where(iota == 3, by2, row)
            row = jnp.where(iota == 4, bos, row)
            plsc.store_scatter(outv, [jnp.full((_L,), t, jnp.int32), iota], row)

        @pl.when(t < _TOPK - 1)
        def _():
            mv, li = sweep(bx1, by1, bx2, by2, barea)
            publish(mv, li, 1 - buf)
            plsc.subcore_barrier()

        return carry

    lax.fori_loop(0, _TOPK, round_body, np.int32(0))

    @pl.when(jnp.logical_and(cid == 0, sid == 0))
    def _():
        pltpu.sync_copy(outv, outh)


@jax.jit
def kernel(x):
    n = x.shape[0]
    npad = ((n + _L * _NS - 1) // (_L * _NS)) * (_L * _NS)
    shard = npad // _NS
    pad = npad - n
    x = x.astype(jnp.float32)
    x1 = jnp.concatenate([x[:, 0], jnp.zeros((pad,), jnp.float32)])
    y1 = jnp.concatenate([x[:, 1], jnp.zeros((pad,), jnp.float32)])
    x2 = jnp.concatenate([x[:, 2], jnp.zeros((pad,), jnp.float32)])
    y2 = jnp.concatenate([x[:, 3], jnp.zeros((pad,), jnp.float32)])
    s = jnp.concatenate([x[:, 4], jnp.full((pad,), _NEG, jnp.float32)])

    mesh = plsc.VectorSubcoreMesh(core_axis_name="c", subcore_axis_name="s",
                                  num_cores=2, num_subcores=_NS)
    f = pl.kernel(
        _nms_body,
        out_type=jax.ShapeDtypeStruct((_TOPK, _L), jnp.float32),
        mesh=mesh,
        compiler_params=pltpu.CompilerParams(needs_layout_passes=False,
                                             use_tc_tiling_on_sc=False),
        scratch_types=[
            pltpu.VMEM((shard,), jnp.float32),
            pltpu.VMEM((shard,), jnp.float32),
            pltpu.VMEM((shard,), jnp.float32),
            pltpu.VMEM((shard,), jnp.float32),
            pltpu.VMEM((shard,), jnp.float32),
            pltpu.VMEM((shard,), jnp.float32),
            pltpu.VMEM((_L,), jnp.float32),
            pltpu.VMEM((_NS, _L), jnp.float32),
            pltpu.VMEM((_TOPK, _L), jnp.float32),
            pltpu.VMEM_SHARED((2, _NS, _L), jnp.float32),
        ],
    )
    out16 = f(x1, y1, x2, y2, s)
    return out16[:, :5]


# speedup vs baseline: 1.7522x; 1.0403x over previous
import jax
import jax.numpy as jnp
import numpy as np
from jax import lax
from jax.experimental import pallas as pl
from jax.experimental.pallas import tpu as pltpu, tpu_sc as plsc

_TOPK = 100
_THRESH = np.float32(0.5)
_NEG = np.float32(-1e30)
_L = 16
_NS = 16
_BIG = np.int32(1 << 30)


def _nms_body(xth, outh,
              x1v, y1v, x2v, y2v, sv, oscv, pubv, candv, outv, shv):
    cid = lax.axis_index("c")
    sid = lax.axis_index("s")
    shard = x1v.shape[0]
    nch = shard // _L
    base = sid * shard
    iota = lax.broadcasted_iota(jnp.int32, (_L,), 0)
    zeros_i = jnp.zeros((_L,), jnp.int32)

    pltpu.sync_copy(xth.at[0, pl.ds(base, shard)], x1v)
    pltpu.sync_copy(xth.at[1, pl.ds(base, shard)], y1v)
    pltpu.sync_copy(xth.at[2, pl.ds(base, shard)], x2v)
    pltpu.sync_copy(xth.at[3, pl.ds(base, shard)], y2v)
    pltpu.sync_copy(xth.at[4, pl.ds(base, shard)], sv)
    pltpu.sync_copy(xth.at[4, pl.ds(base, shard)], oscv)

    def vmax(v):
        for shf in (8, 4, 2, 1):
            p = jnp.take_along_axis(v, iota ^ np.int32(shf), axis=0,
                                    mode="promise_in_bounds")
            v = jnp.maximum(v, p)
        return v

    def vmin_i(v):
        for shf in (8, 4, 2, 1):
            p = jnp.take_along_axis(v, iota ^ np.int32(shf), axis=0,
                                    mode="promise_in_bounds")
            v = jnp.minimum(v, p)
        return v

    def sweep(bx1, by1, bx2, by2, barea):
        def sup_chunk(j, vm):
            sl = pl.ds(j * _L, _L)
            cx1 = x1v[sl]
            cy1 = y1v[sl]
            cx2 = x2v[sl]
            cy2 = y2v[sl]
            cs = sv[sl]
            ca = (cx2 - cx1) * (cy2 - cy1)
            xx1 = jnp.maximum(bx1, cx1)
            yy1 = jnp.maximum(by1, cy1)
            xx2 = jnp.minimum(bx2, cx2)
            yy2 = jnp.minimum(by2, cy2)
            w = jnp.maximum(xx2 - xx1, np.float32(0.0))
            h = jnp.maximum(yy2 - yy1, np.float32(0.0))
            inter = w * h
            denom = barea + ca - inter + np.float32(1e-9)
            sup = (inter + inter) > denom
            ns = jnp.where(sup, _NEG, cs)
            sv[sl] = ns
            return jnp.maximum(vm, ns)

        vm = lax.fori_loop(0, nch, sup_chunk,
                           jnp.full((_L,), -np.float32(np.inf), jnp.float32))
        mv = vmax(vm)

        def idx_chunk(j, iv):
            cs = sv[pl.ds(j * _L, _L)]
            return jnp.minimum(iv, jnp.where(cs == mv, iota + j * _L, _BIG))
        iv = lax.fori_loop(0, nch, idx_chunk, jnp.full((_L,), _BIG, jnp.int32),
                           unroll=8)
        li = vmin_i(iv)
        return mv, li

    def publish(mv, li, buf):
        gx1 = plsc.load_gather(x1v, [li])
        gy1 = plsc.load_gather(y1v, [li])
        gx2 = plsc.load_gather(x2v, [li])
        gy2 = plsc.load_gather(y2v, [li])
        gos = plsc.load_gather(oscv, [li])
        pub = mv
        pub = jnp.where(iota == 1, gx1, pub)
        pub = jnp.where(iota == 2, gy1, pub)
        pub = jnp.where(iota == 3, gx2, pub)
        pub = jnp.where(iota == 4, gy2, pub)
        pub = jnp.where(iota == 5, gos, pub)
        pubv[...] = pub
        pltpu.sync_copy(pubv, shv.at[buf, sid])

    z = jnp.zeros((_L,), jnp.float32)
    mv, li = sweep(z, z, z, z, z)
    publish(mv, li, 0)
    plsc.subcore_barrier()

    def round_body(t, carry):
        buf = lax.rem(t, 2)
        pltpu.sync_copy(shv.at[buf], candv)

        vals = plsc.load_gather(candv, [iota, zeros_i])
        mg = vmax(vals)
        wstar = plsc.all_reduce_ffs(vals == mg)

        def fld(f):
            return plsc.load_gather(candv, [wstar, jnp.full((_L,), f, jnp.int32)])
        bx1 = fld(1)
        by1 = fld(2)
        bx2 = fld(3)
        by2 = fld(4)
        bos = fld(5)
        barea = (bx2 - bx1) * (by2 - by1)

        @pl.when(jnp.logical_and(cid == 0, sid == 0))
        def _():
            row = jnp.where(iota == 0, bx1, jnp.zeros((_L,), jnp.float32))
            row = jnp.where(iota == 1, by1, row)
            row = jnp.where(iota == 2, bx2, row)
            row = jnp.where(iota == 3, by2, row)
            row = jnp.where(iota == 4, bos, row)
            plsc.store_scatter(outv, [jnp.full((_L,), t, jnp.int32), iota], row)

        @pl.when(t < _TOPK - 1)
        def _():
            mv, li = sweep(bx1, by1, bx2, by2, barea)
            publish(mv, li, 1 - buf)
            plsc.subcore_barrier()

        return carry

    lax.fori_loop(0, _TOPK, round_body, np.int32(0))

    @pl.when(jnp.logical_and(cid == 0, sid == 0))
    def _():
        pltpu.sync_copy(outv, outh)


@jax.jit
def kernel(x):
    n = x.shape[0]
    npad = ((n + _L * _NS - 1) // (_L * _NS)) * (_L * _NS)
    shard = npad // _NS
    pad = npad - n
    x = x.astype(jnp.float32)
    padrow = jnp.array([0.0, 0.0, 0.0, 0.0, _NEG], jnp.float32)
    xp = jnp.concatenate([x, jnp.broadcast_to(padrow, (pad, 5))])
    xt = xp.T

    mesh = plsc.VectorSubcoreMesh(core_axis_name="c", subcore_axis_name="s",
                                  num_cores=2, num_subcores=_NS)
    f = pl.kernel(
        _nms_body,
        out_type=jax.ShapeDtypeStruct((_TOPK, _L), jnp.float32),
        mesh=mesh,
        compiler_params=pltpu.CompilerParams(needs_layout_passes=False,
                                             use_tc_tiling_on_sc=False),
        scratch_types=[
            pltpu.VMEM((shard,), jnp.float32),
            pltpu.VMEM((shard,), jnp.float32),
            pltpu.VMEM((shard,), jnp.float32),
            pltpu.VMEM((shard,), jnp.float32),
            pltpu.VMEM((shard,), jnp.float32),
            pltpu.VMEM((shard,), jnp.float32),
            pltpu.VMEM((_L,), jnp.float32),
            pltpu.VMEM((_NS, _L), jnp.float32),
            pltpu.VMEM((_TOPK, _L), jnp.float32),
            pltpu.VMEM_SHARED((2, _NS, _L), jnp.float32),
        ],
    )
    out16 = f(xt)
    return out16[:, :5]


# speedup vs baseline: 1.7963x; 1.0251x over previous
import jax
import jax.numpy as jnp
import numpy as np
from jax import lax
from jax.experimental import pallas as pl
from jax.experimental.pallas import tpu as pltpu, tpu_sc as plsc

_TOPK = 100
_THRESH = np.float32(0.5)
_NEG = np.float32(-1e30)
_L = 16
_NS = 16
_BIG = np.int32(1 << 30)


def _nms_body(xth, outh,
              x1v, y1v, x2v, y2v, sv, oscv, pubv, candv, outv, shv, dsem):
    cid = lax.axis_index("c")
    sid = lax.axis_index("s")
    shard = x1v.shape[0]
    nch = shard // _L
    base = sid * shard
    iota = lax.broadcasted_iota(jnp.int32, (_L,), 0)
    zeros_i = jnp.zeros((_L,), jnp.int32)

    copies = [
        pltpu.async_copy(xth.at[0, pl.ds(base, shard)], x1v, dsem),
        pltpu.async_copy(xth.at[1, pl.ds(base, shard)], y1v, dsem),
        pltpu.async_copy(xth.at[2, pl.ds(base, shard)], x2v, dsem),
        pltpu.async_copy(xth.at[3, pl.ds(base, shard)], y2v, dsem),
        pltpu.async_copy(xth.at[4, pl.ds(base, shard)], sv, dsem),
        pltpu.async_copy(xth.at[4, pl.ds(base, shard)], oscv, dsem),
    ]
    for c in copies:
        c.wait()

    def vmax(v):
        for shf in (8, 4, 2, 1):
            p = jnp.take_along_axis(v, iota ^ np.int32(shf), axis=0,
                                    mode="promise_in_bounds")
            v = jnp.maximum(v, p)
        return v

    def vmin_i(v):
        for shf in (8, 4, 2, 1):
            p = jnp.take_along_axis(v, iota ^ np.int32(shf), axis=0,
                                    mode="promise_in_bounds")
            v = jnp.minimum(v, p)
        return v

    def sweep(bx1, by1, bx2, by2, barea):
        def sup_chunk(j, vm):
            sl = pl.ds(j * _L, _L)
            cx1 = x1v[sl]
            cy1 = y1v[sl]
            cx2 = x2v[sl]
            cy2 = y2v[sl]
            cs = sv[sl]
            ca = (cx2 - cx1) * (cy2 - cy1)
            xx1 = jnp.maximum(bx1, cx1)
            yy1 = jnp.maximum(by1, cy1)
            xx2 = jnp.minimum(bx2, cx2)
            yy2 = jnp.minimum(by2, cy2)
            w = jnp.maximum(xx2 - xx1, np.float32(0.0))
            h = jnp.maximum(yy2 - yy1, np.float32(0.0))
            inter = w * h
            denom = barea + ca - inter + np.float32(1e-9)
            sup = (inter + inter) > denom
            ns = jnp.where(sup, _NEG, cs)
            sv[sl] = ns
            return jnp.maximum(vm, ns)

        vm = lax.fori_loop(0, nch, sup_chunk,
                           jnp.full((_L,), -np.float32(np.inf), jnp.float32))
        mv = vmax(vm)

        def idx_chunk(j, iv):
            cs = sv[pl.ds(j * _L, _L)]
            return jnp.minimum(iv, jnp.where(cs == mv, iota + j * _L, _BIG))
        iv = lax.fori_loop(0, nch, idx_chunk, jnp.full((_L,), _BIG, jnp.int32),
                           unroll=16)
        li = vmin_i(iv)
        return mv, li

    def publish(mv, li, buf):
        gx1 = plsc.load_gather(x1v, [li])
        gy1 = plsc.load_gather(y1v, [li])
        gx2 = plsc.load_gather(x2v, [li])
        gy2 = plsc.load_gather(y2v, [li])
        gos = plsc.load_gather(oscv, [li])
        pub = mv
        pub = jnp.where(iota == 1, gx1, pub)
        pub = jnp.where(iota == 2, gy1, pub)
        pub = jnp.where(iota == 3, gx2, pub)
        pub = jnp.where(iota == 4, gy2, pub)
        pub = jnp.where(iota == 5, gos, pub)
        pubv[...] = pub
        pltpu.sync_copy(pubv, shv.at[buf, sid])

    z = jnp.zeros((_L,), jnp.float32)
    mv, li = sweep(z, z, z, z, z)
    publish(mv, li, 0)
    plsc.subcore_barrier()

    def round_body(t, carry):
        buf = lax.rem(t, 2)
        pltpu.sync_copy(shv.at[buf], candv)

        vals = plsc.load_gather(candv, [iota, zeros_i])
        mg = vmax(vals)
        wstar = plsc.all_reduce_ffs(vals == mg)

        def fld(f):
            return plsc.load_gather(candv, [wstar, jnp.full((_L,), f, jnp.int32)])
        bx1 = fld(1)
        by1 = fld(2)
        bx2 = fld(3)
        by2 = fld(4)
        bos = fld(5)
        barea = (bx2 - bx1) * (by2 - by1)

        @pl.when(jnp.logical_and(cid == 0, sid == 0))
        def _():
            row = jnp.where(iota == 0, bx1, jnp.zeros((_L,), jnp.float32))
            row = jnp.where(iota == 1, by1, row)
            row = jnp.where(iota == 2, bx2, row)
            row = jnp.where(iota == 3, by2, row)
            row = jnp.where(iota == 4, bos, row)
            plsc.store_scatter(outv, [jnp.full((_L,), t, jnp.int32), iota], row)

        @pl.when(t < _TOPK - 1)
        def _():
            mv, li = sweep(bx1, by1, bx2, by2, barea)
            publish(mv, li, 1 - buf)
            plsc.subcore_barrier()

        return carry

    lax.fori_loop(0, _TOPK, round_body, np.int32(0))

    @pl.when(jnp.logical_and(cid == 0, sid == 0))
    def _():
        pltpu.sync_copy(outv, outh)


@jax.jit
def kernel(x):
    n = x.shape[0]
    npad = ((n + _L * _NS - 1) // (_L * _NS)) * (_L * _NS)
    shard = npad // _NS
    pad = npad - n
    x = x.astype(jnp.float32)
    padrow = jnp.array([0.0, 0.0, 0.0, 0.0, _NEG], jnp.float32)
    xp = jnp.concatenate([x, jnp.broadcast_to(padrow, (pad, 5))])
    xt = xp.T

    mesh = plsc.VectorSubcoreMesh(core_axis_name="c", subcore_axis_name="s",
                                  num_cores=2, num_subcores=_NS)
    f = pl.kernel(
        _nms_body,
        out_type=jax.ShapeDtypeStruct((_TOPK, _L), jnp.float32),
        mesh=mesh,
        compiler_params=pltpu.CompilerParams(needs_layout_passes=False,
                                             use_tc_tiling_on_sc=False),
        scratch_types=[
            pltpu.VMEM((shard,), jnp.float32),
            pltpu.VMEM((shard,), jnp.float32),
            pltpu.VMEM((shard,), jnp.float32),
            pltpu.VMEM((shard,), jnp.float32),
            pltpu.VMEM((shard,), jnp.float32),
            pltpu.VMEM((shard,), jnp.float32),
            pltpu.VMEM((_L,), jnp.float32),
            pltpu.VMEM((_NS, _L), jnp.float32),
            pltpu.VMEM((_TOPK, _L), jnp.float32),
            pltpu.VMEM_SHARED((2, _NS, _L), jnp.float32),
            pltpu.SemaphoreType.DMA,
        ],
    )
    out16 = f(xt)
    return out16[:, :5]
